# baseline (speedup 1.0000x reference)
import functools

import jax
import jax.numpy as jnp
from jax import lax
from jax.experimental import pallas as pl
from jax.experimental.pallas import tpu as pltpu
from jax.experimental.pallas import tpu_sc as plsc

_NUM_WORKERS = 32
_CHUNK = 128


def _sc_gather_pair(uid, mid, E_uid, E_mid):
  B = uid.shape[0]
  D = E_uid.shape[1]
  rows_w = B // _NUM_WORKERS
  nck = rows_w // _CHUNK
  uid2 = uid.reshape(B // _CHUNK, _CHUNK)
  mid2 = mid.reshape(B // _CHUNK, _CHUNK)

  mesh = plsc.VectorSubcoreMesh(core_axis_name="c", subcore_axis_name="s")

  @functools.partial(
      pl.kernel,
      out_type=(jax.ShapeDtypeStruct((B, D), jnp.float32),
                jax.ShapeDtypeStruct((B, D), jnp.float32)),
      mesh=mesh,
      compiler_params=pltpu.CompilerParams(use_tc_tiling_on_sc=False),
      scratch_types=[
          pltpu.VMEM((nck, _CHUNK), jnp.int32),
          pltpu.VMEM((nck, _CHUNK), jnp.int32),
          pltpu.VMEM((rows_w, D), jnp.float32),
          pltpu.VMEM((rows_w, D), jnp.float32),
          pltpu.SemaphoreType.DMA,
          pltpu.SemaphoreType.DMA,
      ],
  )
  def gather_kernel(uid_hbm, mid_hbm, eu_hbm, em_hbm, ou_hbm, om_hbm,
                    iu_v, im_v, ru_v, rm_v, sem_u, sem_m):
    wid = lax.axis_index("s") * 2 + lax.axis_index("c")
    base_ck = wid * nck
    pltpu.sync_copy(uid_hbm.at[pl.ds(base_ck, nck)], iu_v)
    pltpu.sync_copy(mid_hbm.at[pl.ds(base_ck, nck)], im_v)
    copies = []
    for j in range(nck):
      copies.append(pltpu.async_copy(
          eu_hbm.at[iu_v.at[j]], ru_v.at[pl.ds(j * _CHUNK, _CHUNK)], sem_u))
      copies.append(pltpu.async_copy(
          em_hbm.at[im_v.at[j]], rm_v.at[pl.ds(j * _CHUNK, _CHUNK)], sem_m))
    for c in copies:
      c.wait()
    base = wid * rows_w
    pltpu.sync_copy(ru_v, ou_hbm.at[pl.ds(base, rows_w)])
    pltpu.sync_copy(rm_v, om_hbm.at[pl.ds(base, rows_w)])

  return gather_kernel(uid2, mid2, E_uid, E_mid)


def _tc_dense(gender, age, job, e_uid, e_mid, E_g, E_a, E_j, att_W, att_b):
  B, D = e_uid.shape
  BM = 1024
  NB = B // BM

  def pad_rows(t, n):
    return jnp.concatenate(
        [t, jnp.zeros((n - t.shape[0], t.shape[1]), t.dtype)], axis=0)

  NG, NA, NJ = 8, 8, 24
  Egp = pad_rows(E_g, NG)
  Eap = pad_rows(E_a, NA)
  Ejp = pad_rows(E_j, NJ)
  g3 = gender.reshape(NB, BM, 1)
  a3 = age.reshape(NB, BM, 1)
  j3 = job.reshape(NB, BM, 1)
  b2 = att_b.reshape(1, D)

  def body(g_ref, a_ref, j_ref, eu_ref, em_ref, eg_ref, ea_ref, ej_ref,
           w_ref, b_ref, o_ref):
    em = em_ref[...]
    eu = eu_ref[...]
    w_top = w_ref[:D, :]
    w_bot = w_ref[D:, :]
    m = jnp.dot(em, w_top, preferred_element_type=jnp.float32) + b_ref[...]

    def attr_cell(idx_ref, table_ref, n):
      ids = idx_ref[0]
      oh = (ids == lax.broadcasted_iota(jnp.int32, (BM, n), 1)
            ).astype(jnp.float32)
      tbl = table_ref[...]
      proj = jnp.dot(tbl, w_bot, preferred_element_type=jnp.float32)
      both = jnp.dot(oh, jnp.concatenate([tbl, proj], axis=1),
                     preferred_element_type=jnp.float32)
      e_att = both[:, :D]
      v = m + both[:, D:]
      v = v - jnp.max(v, axis=1, keepdims=True)
      ex = jnp.exp(v)
      lam = ex / jnp.sum(ex, axis=1, keepdims=True)
      return lam * e_att

    cg = attr_cell(g_ref, eg_ref, NG)
    ca = attr_cell(a_ref, ea_ref, NA)
    cj = attr_cell(j_ref, ej_ref, NJ)
    t = cg + ca + cj
    mn = cg * cg + ca * ca + cj * cj
    p_u = eu * t + 0.5 * (t * t - mn)
    o_ref[...] = jnp.sum(p_u * em, axis=1, keepdims=True)

  return pl.pallas_call(
      body,
      grid=(NB,),
      in_specs=[
          pl.BlockSpec((1, BM, 1), lambda i: (i, 0, 0)),
          pl.BlockSpec((1, BM, 1), lambda i: (i, 0, 0)),
          pl.BlockSpec((1, BM, 1), lambda i: (i, 0, 0)),
          pl.BlockSpec((BM, D), lambda i: (i, 0)),
          pl.BlockSpec((BM, D), lambda i: (i, 0)),
          pl.BlockSpec((NG, D), lambda i: (0, 0)),
          pl.BlockSpec((NA, D), lambda i: (0, 0)),
          pl.BlockSpec((NJ, D), lambda i: (0, 0)),
          pl.BlockSpec((2 * D, D), lambda i: (0, 0)),
          pl.BlockSpec((1, D), lambda i: (0, 0)),
      ],
      out_specs=pl.BlockSpec((BM, 1), lambda i: (i, 0)),
      out_shape=jax.ShapeDtypeStruct((B, 1), jnp.float32),
  )(g3, a3, j3, e_uid, e_mid, Egp, Eap, Ejp, att_W, b2)


def kernel(uid, gender, age, job, mid, E_uid, E_gender, E_age, E_job, E_mid,
           att_W, att_b):
  e_uid, e_mid = _sc_gather_pair(uid, mid, E_uid, E_mid)
  return _tc_dense(gender, age, job, e_uid, e_mid,
                   E_gender, E_age, E_job, att_W, att_b)
